# baseline (speedup 1.0000x reference)
import functools

import jax
import jax.numpy as jnp
from jax import lax
from jax.experimental import pallas as pl
from jax.experimental.pallas import tpu as pltpu
from jax.experimental.pallas import tpu_sc as plsc

EMBED_DIM = 128
SEQ = 50
BATCH = 4096
NUM_WORKERS = 32
BLK = BATCH // NUM_WORKERS
NBUF = 6
MAIN = (SEQ // NBUF) * NBUF


def _sc_embed(idx_t, table):
    mesh = plsc.VectorSubcoreMesh(core_axis_name="c", subcore_axis_name="s")

    @functools.partial(
        pl.kernel,
        mesh=mesh,
        out_type=[
            jax.ShapeDtypeStruct((SEQ, BATCH, EMBED_DIM), jnp.float32),
            jax.ShapeDtypeStruct((SEQ, BATCH), jnp.int32),
        ],
        scratch_types=(
            [pltpu.VMEM((SEQ, BLK), jnp.int32),
             pltpu.VMEM((SEQ, BLK), jnp.int32)]
            + [pltpu.VMEM((BLK, EMBED_DIM), jnp.float32) for _ in range(NBUF)]
            + [pltpu.SemaphoreType.DMA for _ in range(2 * NBUF)]
        ),
    )
    def k(idx_hbm, table_hbm, emb_hbm, mask_hbm, idx_v, mask_v, *bufs_sems):
        rows = bufs_sems[:NBUF]
        gsem = bufs_sems[NBUF:2 * NBUF]
        osem = bufs_sems[2 * NBUF:]
        wid = lax.axis_index("s") * 2 + lax.axis_index("c")
        n0 = wid * BLK

        def gather_start(p, b):
            pltpu.async_copy(table_hbm.at[idx_v.at[p]], rows[b], gsem[b])

        def gather_wait(p, b):
            pltpu.make_async_copy(
                table_hbm.at[idx_v.at[p]], rows[b], gsem[b]
            ).wait()

        def out_start(p, b):
            pltpu.async_copy(rows[b], emb_hbm.at[p, pl.ds(n0, BLK)], osem[b])

        def out_wait(p, b):
            pltpu.make_async_copy(
                rows[b], emb_hbm.at[p, pl.ds(n0, BLK)], osem[b]
            ).wait()

        pltpu.sync_copy(idx_hbm.at[pl.ds(0, SEQ), pl.ds(n0, BLK)], idx_v)
        for b in range(NBUF):
            gather_start(b, b)

        def mask_row(r):
            for c in range(BLK // 16):
                v = idx_v[r, pl.ds(c * 16, 16)]
                mask_v[r, pl.ds(c * 16, 16)] = jnp.minimum(
                    jnp.abs(v), jnp.full((16,), 1, jnp.int32)
                )

        def outer(t, carry):
            for b in range(NBUF):
                p = t * NBUF + b
                mask_row(p)
                gather_wait(p, b)
                out_start(p, b)
                pb = (b - 1) % NBUF
                pp = p + NBUF - 1

                @pl.when((p > 0) & (pp < SEQ))
                def _():
                    out_wait(p - 1, pb)
                    gather_start(pp, pb)

            return carry

        lax.fori_loop(0, SEQ // NBUF, outer, 0)
        for c in range(MAIN, SEQ):
            b = c % NBUF
            mask_row(c)
            gather_wait(c, b)
            out_start(c, b)
            out_wait(c - 1, (b - 1) % NBUF)
        pltpu.sync_copy(mask_v, mask_hbm.at[pl.ds(0, SEQ), pl.ds(n0, BLK)])
        out_wait(SEQ - 1, (SEQ - 1) % NBUF)

    return k(idx_t, table)


def kernel(sen_idx, table):
    idx_t = sen_idx.astype(jnp.int32).T
    emb, mask_i32 = _sc_embed(idx_t, table)
    sen_emb = emb.transpose(1, 0, 2)
    mask = (mask_i32 != 0).T
    return (sen_emb, mask)
